# baseline (speedup 1.0000x reference)
import functools

import jax
import jax.numpy as jnp
from jax import lax
from jax.experimental import pallas as pl
from jax.experimental.pallas import tpu as pltpu
from jax.experimental.pallas import tpu_sc as plsc

VOCAB = 1000000
HIDDEN = 64
NUM_IDS = 4096 * 200

NC = 2
NS = 16
NW = NC * NS
B_PER_W = NUM_IDS // NW

G = 128
K = 4
CHUNK = G * K
N_CHUNKS = B_PER_W // CHUNK


def _make_gather():
    mesh = plsc.VectorSubcoreMesh(core_axis_name="c", subcore_axis_name="s")

    @functools.partial(
        pl.kernel,
        mesh=mesh,
        out_type=jax.ShapeDtypeStruct((NUM_IDS, HIDDEN), jnp.float32),
        scratch_types=[
            pltpu.VMEM((CHUNK,), jnp.int32),
            pltpu.VMEM((CHUNK, HIDDEN), jnp.float32),
            pltpu.SemaphoreType.DMA,
        ],
        compiler_params=pltpu.CompilerParams(use_tc_tiling_on_sc=False),
    )
    def gather_kernel(idx_hbm, table_hbm, out_hbm, idx_v, rows_v, sem):
        wid = lax.axis_index("s") * NC + lax.axis_index("c")
        base = wid * B_PER_W

        def chunk_body(i, _):
            off = base + i * CHUNK
            pltpu.sync_copy(idx_hbm.at[pl.ds(off, CHUNK)], idx_v)
            copies = []
            for j in range(K):
                copies.append(
                    pltpu.async_copy(
                        table_hbm.at[idx_v.at[pl.ds(j * G, G)]],
                        rows_v.at[pl.ds(j * G, G)],
                        sem,
                    )
                )
            for c in copies:
                c.wait()
            pltpu.sync_copy(rows_v, out_hbm.at[pl.ds(off, CHUNK)])
            return 0

        lax.fori_loop(0, N_CHUNKS, chunk_body, 0)

    return gather_kernel


_gather = _make_gather()


def kernel(input_ids, emb):
    idx = input_ids.reshape(-1).astype(jnp.int32)
    h = _gather(idx, emb)
    h = h.reshape(input_ids.shape + (HIDDEN,))
    return (h, h)


# speedup vs baseline: 1.0312x; 1.0312x over previous
import functools

import jax
import jax.numpy as jnp
from jax import lax
from jax.experimental import pallas as pl
from jax.experimental.pallas import tpu as pltpu
from jax.experimental.pallas import tpu_sc as plsc

VOCAB = 1000000
HIDDEN = 64
NUM_IDS = 4096 * 200

NC = 2
NS = 16
NW = NC * NS
B_PER_W = NUM_IDS // NW

G = 128
K = 4
CHUNK = G * K
N_CHUNKS = B_PER_W // CHUNK
NBUF = 2


def _make_gather():
    mesh = plsc.VectorSubcoreMesh(core_axis_name="c", subcore_axis_name="s")

    @functools.partial(
        pl.kernel,
        mesh=mesh,
        out_type=jax.ShapeDtypeStruct((NUM_IDS, HIDDEN), jnp.float32),
        scratch_types=[
            pltpu.VMEM((NBUF * CHUNK,), jnp.int32),
            pltpu.VMEM((NBUF * CHUNK, HIDDEN), jnp.float32),
            pltpu.SemaphoreType.DMA,
        ],
        compiler_params=pltpu.CompilerParams(use_tc_tiling_on_sc=False),
    )
    def gather_kernel(idx_hbm, table_hbm, out_hbm, idx_v, rows_v, gsem):
        wid = lax.axis_index("s") * NC + lax.axis_index("c")
        base = wid * B_PER_W

        def fire(i, slot):
            off = base + i * CHUNK
            voff = slot * CHUNK
            pltpu.sync_copy(idx_hbm.at[pl.ds(off, CHUNK)],
                            idx_v.at[pl.ds(voff, CHUNK)])
            for j in range(K):
                pltpu.async_copy(
                    table_hbm.at[idx_v.at[pl.ds(voff + j * G, G)]],
                    rows_v.at[pl.ds(voff + j * G, G)],
                    gsem,
                )

        def drain_and_store(i, slot):
            off = base + i * CHUNK
            voff = slot * CHUNK
            for j in range(K):
                pltpu.make_async_copy(
                    table_hbm.at[idx_v.at[pl.ds(voff + j * G, G)]],
                    rows_v.at[pl.ds(voff + j * G, G)],
                    gsem,
                ).wait()
            pltpu.sync_copy(rows_v.at[pl.ds(voff, CHUNK)],
                            out_hbm.at[pl.ds(off, CHUNK)])

        fire(0, 0)

        def body(i, _):
            @pl.when(i + 1 < N_CHUNKS)
            def _():
                fire(i + 1, lax.rem(i + 1, NBUF))

            drain_and_store(i, lax.rem(i, NBUF))
            return 0

        lax.fori_loop(0, N_CHUNKS, body, 0)

    return gather_kernel


_gather = _make_gather()


def kernel(input_ids, emb):
    idx = input_ids.reshape(-1).astype(jnp.int32)
    h = _gather(idx, emb)
    h = h.reshape(input_ids.shape + (HIDDEN,))
    return (h, h)
